# Initial kernel scaffold; baseline (speedup 1.0000x reference)
import jax
import jax.numpy as jnp
from jax.experimental import pallas as pl


def kernel(xyz, features, new_xyz, normals, new_normals, nn_idx, mlp0_W, mlp0_b, mlp0_g, mlp0_be, wn0_W, wn0_b, wn0_g, wn0_be, wn1_W, wn1_b, wn1_g, wn1_be, wn2_W, wn2_b, wn2_g, wn2_be, lin_W, lin_b, lnout_g, lnout_b):
    raise NotImplementedError("write your pallas kernel here")


# speedup vs baseline: 3.9418x; 3.9418x over previous
import functools

import jax
import jax.numpy as jnp
from jax import lax
from jax.experimental import pallas as pl
from jax.experimental.pallas import tpu as pltpu
from jax.experimental.pallas import tpu_sc as plsc

_B, _N, _S, _K = 8, 16384, 4096, 16
_BS = _B * _S
_NROWS = _BS * _K
_W = 128
_SBLK = 256
_ROWS = _SBLK * _K
_CH = 128


def _make_sc_gather():
    info = plsc.get_sparse_core_info()
    nw = info.num_cores * info.num_subcores
    chunks = _NROWS // (nw * _CH)
    mesh = plsc.VectorSubcoreMesh(core_axis_name="c", subcore_axis_name="s")

    @functools.partial(
        pl.kernel,
        mesh=mesh,
        out_type=jax.ShapeDtypeStruct((_NROWS, _W), jnp.float32),
        scratch_types=[
            pltpu.VMEM((_CH,), jnp.int32),
            pltpu.VMEM((_CH, _W), jnp.float32),
            pltpu.SemaphoreType.DMA,
        ],
    )
    def gather_k(table, idx, out, idx_v, rows_v, sem):
        wid = lax.axis_index("s") * info.num_cores + lax.axis_index("c")

        def body(i, carry):
            base = (wid * chunks + i) * _CH
            pltpu.sync_copy(idx.at[pl.ds(base, _CH)], idx_v)
            pltpu.async_copy(table.at[idx_v], rows_v, sem).wait()
            pltpu.sync_copy(rows_v, out.at[pl.ds(base, _CH)])
            return carry

        lax.fori_loop(0, chunks, body, 0)

    return gather_k


_sc_gather_cache = []


def _sc_gather(table, idx):
    if not _sc_gather_cache:
        _sc_gather_cache.append(_make_sc_gather())
    return _sc_gather_cache[0](table, idx)


def _rs(x):
    return jnp.sum(x, axis=1, keepdims=True)


def _ln_lrelu(x, g, b):
    m = jnp.mean(x, axis=1, keepdims=True)
    v = jnp.mean((x - m) * (x - m), axis=1, keepdims=True)
    y = (x - m) / jnp.sqrt(v + 1e-5) * g + b
    return jnp.where(y >= 0, y, 0.2 * y)


def _tc_body(g_ref, nn_ref,
             m0w_ref, m0b_ref, m0g_ref, m0e_ref,
             w0w_ref, w0b_ref, w0g_ref, w0e_ref,
             w1w_ref, w1b_ref, w1g_ref, w1e_ref,
             w2w_ref, w2b_ref, w2g_ref, w2e_ref,
             lw_ref, lb_ref, log_ref, lob_ref,
             out_ref):
    G = g_ref[...]
    gf = G[:, 0:64]
    Vg = G[:, 64:72]
    Ng = G[:, 72:80]
    NNv = nn_ref[...]
    P = NNv[:, 0:8]
    A = NNv[:, 8:16]

    D = Vg - P
    t8 = jnp.sqrt(_rs(D * D))
    Rh = D * (1.0 / jnp.maximum(t8, 1e-12))
    t2 = _rs(A * Rh)
    V = A - t2 * Rh
    V = V * (1.0 / jnp.maximum(jnp.sqrt(_rs(V * V)), 1e-12))

    dx, dy, dz = D[:, 0:1], D[:, 1:2], D[:, 2:3]
    rx, ry, rz = Rh[:, 0:1], Rh[:, 1:2], Rh[:, 2:3]
    vx, vy, vz = V[:, 0:1], V[:, 1:2], V[:, 2:3]
    ax, ay, az = A[:, 0:1], A[:, 1:2], A[:, 2:3]
    gnx, gny, gnz = Ng[:, 0:1], Ng[:, 1:2], Ng[:, 2:3]

    cx = ry * vz - rz * vy
    cy = rz * vx - rx * vz
    cz = rx * vy - ry * vx
    cn = jnp.maximum(jnp.sqrt(cx * cx + cy * cy + cz * cz), 1e-12)
    t5 = (cx * gnx + cy * gny + cz * gnz) / cn
    ex = gny * az - gnz * ay
    ey = gnz * ax - gnx * az
    ez = gnx * ay - gny * ax
    t7 = dx * ex + dy * ey + dz * ez

    t1 = _rs(Ng * A)
    t3 = _rs(Rh * Ng)
    t4 = _rs(V * Ng)
    t6 = _rs(D * A)

    wni = jnp.concatenate(
        [t1, t2, t3, t4, t5, t6, t7, t8, dx, dy, dz,
         jnp.zeros((_ROWS, 5), jnp.float32)], axis=1)

    rel = _ln_lrelu(
        jnp.dot(wni, m0w_ref[...], preferred_element_type=jnp.float32)
        + m0b_ref[...], m0g_ref[...], m0e_ref[...])
    w_ = _ln_lrelu(
        jnp.dot(wni, w0w_ref[...], preferred_element_type=jnp.float32)
        + w0b_ref[...], w0g_ref[...], w0e_ref[...])
    w_ = _ln_lrelu(
        jnp.dot(w_, w1w_ref[...], preferred_element_type=jnp.float32)
        + w1b_ref[...], w1g_ref[...], w1e_ref[...])
    w_ = _ln_lrelu(
        jnp.dot(w_, w2w_ref[...], preferred_element_type=jnp.float32)
        + w2b_ref[...], w2g_ref[...], w2e_ref[...])

    gfull = jnp.concatenate([gf, rel], axis=1)

    accs = []
    for m in range(_K):
        Pm = gfull * w_[:, m:m + 1]
        accs.append(jnp.sum(Pm.reshape(_SBLK, _K, 128), axis=1))
    Y = jnp.concatenate(accs, axis=1)

    o = jnp.dot(Y, lw_ref[...], preferred_element_type=jnp.float32) + lb_ref[...]
    mo = jnp.mean(o, axis=1, keepdims=True)
    vo = jnp.mean((o - mo) * (o - mo), axis=1, keepdims=True)
    o = (o - mo) / jnp.sqrt(vo + 1e-5) * log_ref[...] + lob_ref[...]
    out_ref[...] = jnp.where(o >= 0, o, 0.2 * o)


def _pad_last(x, w):
    return jnp.concatenate(
        [x, jnp.zeros(x.shape[:-1] + (w - x.shape[-1],), x.dtype)], axis=-1)


def kernel(xyz, features, new_xyz, normals, new_normals, nn_idx,
           mlp0_W, mlp0_b, mlp0_g, mlp0_be,
           wn0_W, wn0_b, wn0_g, wn0_be,
           wn1_W, wn1_b, wn1_g, wn1_be,
           wn2_W, wn2_b, wn2_g, wn2_be,
           lin_W, lin_b, lnout_g, lnout_b):
    table = jnp.concatenate(
        [features, _pad_last(xyz, 8), _pad_last(normals, 8),
         jnp.zeros((_B, _N, _W - 80), jnp.float32)],
        axis=-1).reshape(_B * _N, _W)
    flat_idx = (nn_idx
                + (jnp.arange(_B, dtype=jnp.int32) * _N)[:, None, None]
                ).reshape(_NROWS)
    NN = jnp.concatenate(
        [_pad_last(new_xyz, 8), _pad_last(new_normals, 8)], axis=-1)
    NN = jnp.broadcast_to(NN[:, :, None, :], (_B, _S, _K, 16)).reshape(_NROWS, 16)

    m0w = _pad_last(mlp0_W.T, 16).T
    w0w = _pad_last(wn0_W.T, 16).T
    lwp = lin_W.reshape(128, 16, 128).transpose(1, 0, 2).reshape(2048, 128)

    row2 = lambda a: a.reshape(1, -1)

    G = _sc_gather(table, flat_idx)

    nblk = _BS // _SBLK
    full = lambda shape: pl.BlockSpec(shape, lambda i: (0, 0))
    out = pl.pallas_call(
        _tc_body,
        grid=(nblk,),
        in_specs=[
            pl.BlockSpec((_ROWS, _W), lambda i: (i, 0)),
            pl.BlockSpec((_ROWS, 16), lambda i: (i, 0)),
            full((16, 64)), full((1, 64)), full((1, 64)), full((1, 64)),
            full((16, 8)), full((1, 8)), full((1, 8)), full((1, 8)),
            full((8, 8)), full((1, 8)), full((1, 8)), full((1, 8)),
            full((8, 16)), full((1, 16)), full((1, 16)), full((1, 16)),
            full((2048, 128)), full((1, 128)), full((1, 128)), full((1, 128)),
        ],
        out_specs=pl.BlockSpec((_SBLK, 128), lambda i: (i, 0)),
        out_shape=jax.ShapeDtypeStruct((_BS, 128), jnp.float32),
    )(G, NN,
      m0w, row2(mlp0_b), row2(mlp0_g), row2(mlp0_be),
      w0w, row2(wn0_b), row2(wn0_g), row2(wn0_be),
      wn1_W, row2(wn1_b), row2(wn1_g), row2(wn1_be),
      wn2_W, row2(wn2_b), row2(wn2_g), row2(wn2_be),
      lwp, row2(lin_b), row2(lnout_g), row2(lnout_b))

    return out.reshape(_B, _S, 128)


# speedup vs baseline: 3.9731x; 1.0079x over previous
import functools

import jax
import jax.numpy as jnp
from jax import lax
from jax.experimental import pallas as pl
from jax.experimental.pallas import tpu as pltpu
from jax.experimental.pallas import tpu_sc as plsc

_B, _N, _S, _K = 8, 16384, 4096, 16
_BS = _B * _S
_NROWS = _BS * _K
_W = 128
_SBLK = 256
_ROWS = _SBLK * _K
_CH = 128


def _make_sc_gather():
    info = plsc.get_sparse_core_info()
    nw = info.num_cores * info.num_subcores
    chunks = _NROWS // (nw * _CH)
    mesh = plsc.VectorSubcoreMesh(core_axis_name="c", subcore_axis_name="s")

    @functools.partial(
        pl.kernel,
        mesh=mesh,
        out_type=jax.ShapeDtypeStruct((_NROWS, _W), jnp.float32),
        scratch_types=[
            pltpu.VMEM((_CH,), jnp.int32),
            pltpu.VMEM((_CH, _W), jnp.float32),
            pltpu.SemaphoreType.DMA,
        ],
    )
    def gather_k(table, idx, out, idx_v, rows_v, sem):
        wid = lax.axis_index("s") * info.num_cores + lax.axis_index("c")

        def body(i, carry):
            base = (wid * chunks + i) * _CH
            pltpu.sync_copy(idx.at[pl.ds(base, _CH)], idx_v)
            pltpu.async_copy(table.at[idx_v], rows_v, sem).wait()
            pltpu.sync_copy(rows_v, out.at[pl.ds(base, _CH)])
            return carry

        lax.fori_loop(0, chunks, body, 0)

    return gather_k


_sc_gather_cache = []


def _sc_gather(table, idx):
    if not _sc_gather_cache:
        _sc_gather_cache.append(_make_sc_gather())
    return _sc_gather_cache[0](table, idx)


def _rs(x):
    return jnp.sum(x, axis=1, keepdims=True)


def _ln_lrelu(x, g, b):
    m = jnp.mean(x, axis=1, keepdims=True)
    v = jnp.mean((x - m) * (x - m), axis=1, keepdims=True)
    y = (x - m) / jnp.sqrt(v + 1e-5) * g + b
    return jnp.where(y >= 0, y, 0.2 * y)


def _tc_body(g_ref, nn_ref,
             m0w_ref, m0b_ref, m0g_ref, m0e_ref,
             w0w_ref, w0b_ref, w0g_ref, w0e_ref,
             w1w_ref, w1b_ref, w1g_ref, w1e_ref,
             w2w_ref, w2b_ref, w2g_ref, w2e_ref,
             lw_ref, lb_ref, log_ref, lob_ref,
             out_ref):
    G3 = g_ref[...]
    gf = G3[:, :, 0:64].reshape(_ROWS, 64)
    kl = lambda c: G3[:, :, c]
    gx, gy, gz = kl(64), kl(65), kl(66)
    gnx, gny, gnz = kl(72), kl(73), kl(74)
    NNv = nn_ref[...]
    nx, ny, nz = NNv[:, 0:1], NNv[:, 1:2], NNv[:, 2:3]
    ax, ay, az = NNv[:, 8:9], NNv[:, 9:10], NNv[:, 10:11]

    dx, dy, dz = gx - nx, gy - ny, gz - nz
    t8 = jnp.sqrt(dx * dx + dy * dy + dz * dz)
    rinv = 1.0 / jnp.maximum(t8, 1e-12)
    rx, ry, rz = dx * rinv, dy * rinv, dz * rinv
    t2 = ax * rx + ay * ry + az * rz
    vx, vy, vz = ax - t2 * rx, ay - t2 * ry, az - t2 * rz
    vinv = 1.0 / jnp.maximum(jnp.sqrt(vx * vx + vy * vy + vz * vz), 1e-12)
    vx, vy, vz = vx * vinv, vy * vinv, vz * vinv

    cx = ry * vz - rz * vy
    cy = rz * vx - rx * vz
    cz = rx * vy - ry * vx
    cinv = 1.0 / jnp.maximum(jnp.sqrt(cx * cx + cy * cy + cz * cz), 1e-12)
    t5 = (cx * gnx + cy * gny + cz * gnz) * cinv
    ex = gny * az - gnz * ay
    ey = gnz * ax - gnx * az
    ez = gnx * ay - gny * ax
    t7 = dx * ex + dy * ey + dz * ez

    t1 = gnx * ax + gny * ay + gnz * az
    t3 = rx * gnx + ry * gny + rz * gnz
    t4 = vx * gnx + vy * gny + vz * gnz
    t6 = dx * ax + dy * ay + dz * az

    z = jnp.zeros((_SBLK, _K), jnp.float32)
    wni = jnp.stack(
        [t1, t2, t3, t4, t5, t6, t7, t8, dx, dy, dz, z, z, z, z, z],
        axis=-1).reshape(_ROWS, 16)

    rel = _ln_lrelu(
        jnp.dot(wni, m0w_ref[...], preferred_element_type=jnp.float32)
        + m0b_ref[...], m0g_ref[...], m0e_ref[...])
    w_ = _ln_lrelu(
        jnp.dot(wni, w0w_ref[...], preferred_element_type=jnp.float32)
        + w0b_ref[...], w0g_ref[...], w0e_ref[...])
    w_ = _ln_lrelu(
        jnp.dot(w_, w1w_ref[...], preferred_element_type=jnp.float32)
        + w1b_ref[...], w1g_ref[...], w1e_ref[...])
    w_ = _ln_lrelu(
        jnp.dot(w_, w2w_ref[...], preferred_element_type=jnp.float32)
        + w2b_ref[...], w2g_ref[...], w2e_ref[...])

    gfull = jnp.concatenate([gf, rel], axis=1)

    accs = []
    for m in range(_K):
        Pm = gfull * w_[:, m:m + 1]
        accs.append(jnp.sum(Pm.reshape(_SBLK, _K, 128), axis=1))
    Y = jnp.concatenate(accs, axis=1)

    o = jnp.dot(Y, lw_ref[...], preferred_element_type=jnp.float32) + lb_ref[...]
    mo = jnp.mean(o, axis=1, keepdims=True)
    vo = jnp.mean((o - mo) * (o - mo), axis=1, keepdims=True)
    o = (o - mo) / jnp.sqrt(vo + 1e-5) * log_ref[...] + lob_ref[...]
    out_ref[...] = jnp.where(o >= 0, o, 0.2 * o)


def _pad_last(x, w):
    return jnp.concatenate(
        [x, jnp.zeros(x.shape[:-1] + (w - x.shape[-1],), x.dtype)], axis=-1)


def kernel(xyz, features, new_xyz, normals, new_normals, nn_idx,
           mlp0_W, mlp0_b, mlp0_g, mlp0_be,
           wn0_W, wn0_b, wn0_g, wn0_be,
           wn1_W, wn1_b, wn1_g, wn1_be,
           wn2_W, wn2_b, wn2_g, wn2_be,
           lin_W, lin_b, lnout_g, lnout_b):
    table = jnp.concatenate(
        [features, _pad_last(xyz, 8), _pad_last(normals, 8),
         jnp.zeros((_B, _N, _W - 80), jnp.float32)],
        axis=-1).reshape(_B * _N, _W)
    flat_idx = (nn_idx
                + (jnp.arange(_B, dtype=jnp.int32) * _N)[:, None, None]
                ).reshape(_NROWS)
    NN = jnp.concatenate(
        [_pad_last(new_xyz, 8), _pad_last(new_normals, 8)],
        axis=-1).reshape(_BS, 16)

    m0w = _pad_last(mlp0_W.T, 16).T
    w0w = _pad_last(wn0_W.T, 16).T
    lwp = lin_W.reshape(128, 16, 128).transpose(1, 0, 2).reshape(2048, 128)

    row2 = lambda a: a.reshape(1, -1)

    G = _sc_gather(table, flat_idx)

    nblk = _BS // _SBLK
    full = lambda shape: pl.BlockSpec(shape, lambda i: (0, 0))
    out = pl.pallas_call(
        _tc_body,
        grid=(nblk,),
        in_specs=[
            pl.BlockSpec((_SBLK, _K, _W), lambda i: (i, 0, 0)),
            pl.BlockSpec((_SBLK, 16), lambda i: (i, 0)),
            full((16, 64)), full((1, 64)), full((1, 64)), full((1, 64)),
            full((16, 8)), full((1, 8)), full((1, 8)), full((1, 8)),
            full((8, 8)), full((1, 8)), full((1, 8)), full((1, 8)),
            full((8, 16)), full((1, 16)), full((1, 16)), full((1, 16)),
            full((2048, 128)), full((1, 128)), full((1, 128)), full((1, 128)),
        ],
        out_specs=pl.BlockSpec((_SBLK, 128), lambda i: (i, 0)),
        out_shape=jax.ShapeDtypeStruct((_BS, 128), jnp.float32),
    )(G.reshape(_BS, _K, _W), NN,
      m0w, row2(mlp0_b), row2(mlp0_g), row2(mlp0_be),
      w0w, row2(wn0_b), row2(wn0_g), row2(wn0_be),
      wn1_W, row2(wn1_b), row2(wn1_g), row2(wn1_be),
      wn2_W, row2(wn2_b), row2(wn2_g), row2(wn2_be),
      lwp, row2(lin_b), row2(lnout_g), row2(lnout_b))

    return out.reshape(_B, _S, 128)
